# baseline (speedup 1.0000x reference)
import jax
import jax.numpy as jnp
from jax.experimental import pallas as pl
from jax.experimental.pallas import tpu as pltpu

_FILTERS = 64
_C = 64
_D = _H = _W = 32
_N = _D * _H * _W
_PY, _PX = 34, 34
_PLANE = _PY * _PX
_MARGIN = 48
_NROW = 39400
_WROWS = 2 * _PLANE
_TB = 2048
_G = 16

_OFFS = tuple((kk // 9 - 1) * _PLANE + ((kk // 3) % 3 - 1) * _PY + (kk % 3 - 1)
              for kk in range(27))


def _body(feat_hbm, w_ref, b_ref, idx_ref, out_ref, fz_ref, fscr_ref, acc_ref, sem):
    g = pl.program_id(0)

    @pl.when(g == 0)
    def _build():
        fscr_ref[...] = jnp.zeros((_NROW, _C), jnp.float32)
        cp = pltpu.make_async_copy(feat_hbm, fz_ref, sem)
        cp.start()
        cp.wait()

        def scatter(i, carry):
            z = i // 32
            y = i - z * 32
            dst = (_MARGIN + _PLANE + _PY + 1) + z * _PLANE + y * _PY
            fscr_ref[pl.ds(dst, 32), :] = fz_ref[pl.ds(i * 32, 32), :]
            return carry

        jax.lax.fori_loop(0, 1024, scatter, 0)

    ws = _MARGIN + (2 * g + 1) * _PLANE
    acc = None
    for kk, off in enumerate(_OFFS):
        part = jnp.dot(fscr_ref[pl.ds(ws + off, _WROWS), :], w_ref[kk],
                       preferred_element_type=jnp.float32)
        acc = part if acc is None else acc + part
    acc_ref[...] = acc

    def gather(c, carry):
        zl = c // 32
        y = c - zl * 32
        loc = zl * _PLANE + (y + 1) * _PY + 1
        m = (idx_ref[pl.ds(c * 32, 32), :] != 0).astype(jnp.float32)
        out_ref[pl.ds(c * 32, 32), :] = (
            (acc_ref[pl.ds(loc, 32), :] + b_ref[...]) * m)
        return carry

    jax.lax.fori_loop(0, 64, gather, 0)


def kernel(feat, index, w, b):
    out = pl.pallas_call(
        _body,
        grid=(_G,),
        in_specs=[
            pl.BlockSpec(memory_space=pltpu.MemorySpace.HBM),
            pl.BlockSpec((27, _C, _FILTERS), lambda g: (0, 0, 0)),
            pl.BlockSpec((1, _FILTERS), lambda g: (0, 0)),
            pl.BlockSpec((_TB, 1), lambda g: (g, 0)),
        ],
        out_specs=pl.BlockSpec((_TB, _FILTERS), lambda g: (g, 0)),
        out_shape=jax.ShapeDtypeStruct((_N, _FILTERS), jnp.float32),
        scratch_shapes=[
            pltpu.VMEM((_N, _C), jnp.float32),
            pltpu.VMEM((_NROW, _C), jnp.float32),
            pltpu.VMEM((_WROWS, _FILTERS), jnp.float32),
            pltpu.SemaphoreType.DMA,
        ],
    )(feat.reshape(_N, _C), w.reshape(27, _C, _FILTERS),
      b.reshape(1, _FILTERS), index.reshape(_N, 1))
    return out.reshape(1, _D, _H, _W, _FILTERS)


# speedup vs baseline: 2.2512x; 2.2512x over previous
import jax
import jax.numpy as jnp
from jax.experimental import pallas as pl

_FILTERS = 64
_C = 64
_D = _H = _W = 32
_N = _D * _H * _W
_ZPAD = 1088
_NROW = _N + 2 * _ZPAD
_TB = 2048
_G = 16


def _body(fz_ref, w_ref, b_ref, idx_ref, out_ref):
    g = pl.program_id(0)
    base = _ZPAD + g * _TB
    i = jax.lax.broadcasted_iota(jnp.int32, (_TB, 1), 0) + g * _TB
    x = jnp.bitwise_and(i, 31)
    y = jnp.bitwise_and(jax.lax.shift_right_logical(i, 5), 31)
    acc = None
    for dy in (-1, 0, 1):
        my = None if dy == 0 else (y >= 1 if dy < 0 else y <= 30)
        for dx in (-1, 0, 1):
            mx = None if dx == 0 else (x >= 1 if dx < 0 else x <= 30)
            cond = my if mx is None else (mx if my is None else jnp.logical_and(mx, my))
            part = None
            for dz in (-1, 0, 1):
                kk = (dz + 1) * 9 + (dy + 1) * 3 + (dx + 1)
                off = dz * 1024 + dy * 32 + dx
                p = jnp.dot(fz_ref[pl.ds(base + off, _TB), :], w_ref[kk],
                            preferred_element_type=jnp.float32)
                part = p if part is None else part + p
            if cond is not None:
                part = part * cond.astype(jnp.float32)
            acc = part if acc is None else acc + part
    act = (idx_ref[...] != 0).astype(jnp.float32)
    out_ref[...] = (acc + b_ref[...]) * act


def kernel(feat, index, w, b):
    fz = jnp.pad(feat.reshape(_N, _C), ((_ZPAD, _ZPAD), (0, 0)))
    out = pl.pallas_call(
        _body,
        grid=(_G,),
        in_specs=[
            pl.BlockSpec((_NROW, _C), lambda g: (0, 0)),
            pl.BlockSpec((27, _C, _FILTERS), lambda g: (0, 0, 0)),
            pl.BlockSpec((1, _FILTERS), lambda g: (0, 0)),
            pl.BlockSpec((_TB, 1), lambda g: (g, 0)),
        ],
        out_specs=pl.BlockSpec((_TB, _FILTERS), lambda g: (g, 0)),
        out_shape=jax.ShapeDtypeStruct((_N, _FILTERS), jnp.float32),
    )(fz, w.reshape(27, _C, _FILTERS), b.reshape(1, _FILTERS),
      index.reshape(_N, 1))
    return out.reshape(1, _D, _H, _W, _FILTERS)
